# baseline (speedup 1.0000x reference)
import functools

import jax
import jax.numpy as jnp
from jax.experimental import pallas as pl
from jax.experimental.pallas import tpu as pltpu

_DYN = 96
_STAT = 32
_U = 8
_NOBS = 13
_TOTAL_IN = _DYN + _STAT + 1
_HZ = 128
_H1 = 200
_H2 = 200
_LANE = 128

_TB = 256


def _fused_body(x_ref, v1_ref,
                w1_ref, b1_ref, w2_ref, b2_ref, w3_ref, b3_ref,
                wab_ref, bab_ref, wcd_ref, bcd_ref,
                z_ref, y_ref):
    bf16 = jnp.bfloat16
    x = x_ref[...]
    h = jnp.maximum(
        jnp.dot(x, w1_ref[...], preferred_element_type=jnp.float32)
        + b1_ref[...], 0.0)
    h = jnp.maximum(
        jnp.dot(h.astype(bf16), w2_ref[...], preferred_element_type=jnp.float32)
        + b2_ref[...], 0.0)
    hz = (jnp.dot(h.astype(bf16), w3_ref[...], preferred_element_type=jnp.float32)
          + b3_ref[...])

    hzb = hz.astype(bf16)
    ab = (jnp.dot(hzb, wab_ref[...], preferred_element_type=jnp.float32)
          + bab_ref[...])
    ab3 = ab.reshape(_TB, _DYN, _LANE)
    v1 = v1_ref[...]
    z_next = jnp.sum(ab3 * v1[:, None, :], axis=2)

    cd = (jnp.dot(hzb, wcd_ref[...], preferred_element_type=jnp.float32)
          + bcd_ref[...])
    cd3 = cd.reshape(_TB, _NOBS, _LANE)
    v2 = jnp.concatenate([z_next, v1[:, _DYN:]], axis=1)
    yt = jnp.sum(cd3 * v2[:, None, :], axis=2)

    z_ref[...] = z_next
    y_ref[...] = yt


@jax.jit
def kernel(z_dyn, z_static, dt, ut, W1, b1, W2, b2, W3, b3,
           Wa, ba, Wb, bb, Wc, bc, Wd, bd):
    B = z_dyn.shape[0]
    f32 = jnp.float32

    x = jnp.concatenate([z_dyn, z_static, dt], axis=-1)
    pad_b = jnp.zeros((B, _LANE - _DYN - _U), dtype=f32)
    v1 = jnp.concatenate([z_dyn, ut * dt, pad_b], axis=-1)

    def interleave(Wx, bx, Wy, by, rows):
        padw = jnp.zeros((_HZ, rows, _LANE - _DYN - _U), dtype=f32)
        W = jnp.concatenate(
            [Wx.reshape(_HZ, rows, _DYN), Wy.reshape(_HZ, rows, _U), padw],
            axis=2).reshape(_HZ, rows * _LANE)
        padb = jnp.zeros((rows, _LANE - _DYN - _U), dtype=f32)
        bvec = jnp.concatenate(
            [bx.reshape(rows, _DYN), by.reshape(rows, _U), padb],
            axis=1).reshape(1, rows * _LANE)
        return W, bvec

    Wab, bab = interleave(Wa, ba, Wb, bb, _DYN)
    Wcd, bcd = interleave(Wc, bc, Wd, bd, _NOBS)

    bf16 = jnp.bfloat16
    x = x.astype(bf16)
    W1b, W2b, W3b = W1.astype(bf16), W2.astype(bf16), W3.astype(bf16)
    Wab, Wcd = Wab.astype(bf16), Wcd.astype(bf16)

    grid = (B // _TB,)
    row_spec = lambda n: pl.BlockSpec((_TB, n), lambda i: (i, 0))
    w_spec = lambda shp: pl.BlockSpec(shp, lambda i: (0, 0))

    z_next, yt = pl.pallas_call(
        _fused_body,
        grid=grid,
        in_specs=[
            row_spec(_TOTAL_IN),
            row_spec(_LANE),
            w_spec((_TOTAL_IN, _H1)), w_spec((1, _H1)),
            w_spec((_H1, _H2)), w_spec((1, _H2)),
            w_spec((_H2, _HZ)), w_spec((1, _HZ)),
            w_spec((_HZ, _DYN * _LANE)), w_spec((1, _DYN * _LANE)),
            w_spec((_HZ, _NOBS * _LANE)), w_spec((1, _NOBS * _LANE)),
        ],
        out_specs=[row_spec(_DYN), row_spec(_NOBS)],
        out_shape=[
            jax.ShapeDtypeStruct((B, _DYN), f32),
            jax.ShapeDtypeStruct((B, _NOBS), f32),
        ],
        compiler_params=pltpu.CompilerParams(
            dimension_semantics=("arbitrary",)),
    )(x, v1, W1b, b1.reshape(1, -1), W2b, b2.reshape(1, -1),
      W3b, b3.reshape(1, -1), Wab, bab, Wcd, bcd)

    return (z_next, yt)


# speedup vs baseline: 1.4000x; 1.4000x over previous
import jax
import jax.numpy as jnp
from jax.experimental import pallas as pl
from jax.experimental.pallas import tpu as pltpu

_DYN = 96
_STAT = 32
_U = 8
_NOBS = 13
_TOTAL_IN = _DYN + _STAT + 1
_HZ = 128
_H1 = 200
_H2 = 200
_LANE = 128

_TB = 256


def _fused_body(x_ref, v1_ref, ushift_ref,
                w1_ref, b1_ref, w2_ref, b2_ref, w3_ref, b3_ref,
                wab_ref, bab_ref, wcd_ref, bcd_ref,
                sab_ref, scd_ref,
                z_ref, y_ref):
    bf16 = jnp.bfloat16
    f32 = jnp.float32
    x = x_ref[...]
    h = jnp.maximum(
        jnp.dot(x, w1_ref[...], preferred_element_type=f32) + b1_ref[...],
        0.0)
    h = jnp.maximum(
        jnp.dot(h, w2_ref[...], preferred_element_type=f32) + b2_ref[...],
        0.0)
    hz = jnp.dot(h, w3_ref[...], preferred_element_type=f32) + b3_ref[...]
    hzb = hz.astype(bf16)

    ab = (jnp.dot(hzb, wab_ref[...], preferred_element_type=f32)
          + bab_ref[...])
    v1 = v1_ref[...]
    v1rep = jnp.concatenate([v1] * _DYN, axis=1)
    p_ab = (ab * v1rep).astype(bf16)
    zfull = jnp.dot(p_ab, sab_ref[...],
                    preferred_element_type=f32)

    v2 = zfull + ushift_ref[...]
    cd = (jnp.dot(hzb, wcd_ref[...], preferred_element_type=f32)
          + bcd_ref[...])
    v2rep = jnp.concatenate([v2] * _NOBS, axis=1)
    p_cd = (cd * v2rep).astype(bf16)
    yt = jnp.dot(p_cd, scd_ref[...],
                 preferred_element_type=f32)

    z_ref[...] = zfull[:, :_DYN]
    y_ref[...] = yt


@jax.jit
def kernel(z_dyn, z_static, dt, ut, W1, b1, W2, b2, W3, b3,
           Wa, ba, Wb, bb, Wc, bc, Wd, bd):
    B = z_dyn.shape[0]
    f32 = jnp.float32
    bf16 = jnp.bfloat16

    x = jnp.concatenate([z_dyn, z_static, dt], axis=-1)
    utdt = ut * dt
    pad_b = jnp.zeros((B, _LANE - _DYN - _U), dtype=f32)
    v1 = jnp.concatenate([z_dyn, utdt, pad_b], axis=-1)
    ushift = jnp.concatenate(
        [jnp.zeros((B, _DYN), dtype=f32), utdt, pad_b], axis=-1)

    def interleave(Wx, bx, Wy, by, rows):
        padw = jnp.zeros((_HZ, rows, _LANE - _DYN - _U), dtype=f32)
        W = jnp.concatenate(
            [Wx.reshape(_HZ, rows, _DYN), Wy.reshape(_HZ, rows, _U), padw],
            axis=2).reshape(_HZ, rows * _LANE)
        padb = jnp.zeros((rows, _LANE - _DYN - _U), dtype=f32)
        bvec = jnp.concatenate(
            [bx.reshape(rows, _DYN), by.reshape(rows, _U), padb],
            axis=1).reshape(1, rows * _LANE)
        return W.astype(bf16), bvec

    Wab, bab = interleave(Wa, ba, Wb, bb, _DYN)
    Wcd, bcd = interleave(Wc, bc, Wd, bd, _NOBS)

    sab = (jnp.arange(_DYN * _LANE)[:, None] // _LANE
           == jnp.arange(_LANE)[None, :]).astype(bf16)
    scd = (jnp.arange(_NOBS * _LANE)[:, None] // _LANE
           == jnp.arange(_NOBS)[None, :]).astype(bf16)

    grid = (B // _TB,)
    row_spec = lambda n: pl.BlockSpec((_TB, n), lambda i: (i, 0))
    w_spec = lambda shp: pl.BlockSpec(shp, lambda i: (0,) * len(shp))

    z_next, yt = pl.pallas_call(
        _fused_body,
        grid=grid,
        in_specs=[
            row_spec(_TOTAL_IN),
            row_spec(_LANE),
            row_spec(_LANE),
            w_spec((_TOTAL_IN, _H1)), w_spec((1, _H1)),
            w_spec((_H1, _H2)), w_spec((1, _H2)),
            w_spec((_H2, _HZ)), w_spec((1, _HZ)),
            w_spec((_HZ, _DYN * _LANE)), w_spec((1, _DYN * _LANE)),
            w_spec((_HZ, _NOBS * _LANE)), w_spec((1, _NOBS * _LANE)),
            w_spec((_DYN * _LANE, _LANE)),
            w_spec((_NOBS * _LANE, _NOBS)),
        ],
        out_specs=[row_spec(_DYN), row_spec(_NOBS)],
        out_shape=[
            jax.ShapeDtypeStruct((B, _DYN), f32),
            jax.ShapeDtypeStruct((B, _NOBS), f32),
        ],
        compiler_params=pltpu.CompilerParams(
            dimension_semantics=("arbitrary",)),
    )(x, v1, ushift, W1, b1.reshape(1, -1), W2, b2.reshape(1, -1),
      W3, b3.reshape(1, -1), Wab, bab, Wcd, bcd, sab, scd)

    return (z_next, yt)


# speedup vs baseline: 1.4198x; 1.0141x over previous
import jax
import jax.numpy as jnp
from jax.experimental import pallas as pl
from jax.experimental.pallas import tpu as pltpu

_DYN = 96
_STAT = 32
_U = 8
_NOBS = 13
_HZ = 128
_H1 = 200
_H2 = 200
_LANE = 128
_PAD = _LANE - _DYN - _U

_TB = 256


def _fused_body(zd_ref, zs_ref, dt_ref, ut_ref,
                w1d_ref, w1s_ref, w1t_ref, b1_ref,
                w2_ref, b2_ref, w3_ref, b3_ref,
                wab_ref, bab_ref, wcd_ref, bcd_ref,
                sab_ref, scd_ref,
                z_ref, y_ref):
    bf16 = jnp.bfloat16
    f32 = jnp.float32
    zd = zd_ref[...]
    dt = dt_ref[...]
    h = (jnp.dot(zd.astype(bf16), w1d_ref[...], preferred_element_type=f32)
         + jnp.dot(zs_ref[...].astype(bf16), w1s_ref[...],
                   preferred_element_type=f32)
         + dt * w1t_ref[...] + b1_ref[...])
    h = jnp.maximum(h, 0.0)
    h = jnp.maximum(
        jnp.dot(h.astype(bf16), w2_ref[...], preferred_element_type=f32)
        + b2_ref[...], 0.0)
    hz = (jnp.dot(h.astype(bf16), w3_ref[...], preferred_element_type=f32)
          + b3_ref[...])
    hzb = hz.astype(bf16)

    utdt = ut_ref[...] * dt
    zpad = jnp.zeros((_TB, _PAD), dtype=f32)
    v1 = jnp.concatenate([zd, utdt, zpad], axis=1)
    ushift = jnp.concatenate(
        [jnp.zeros((_TB, _DYN), dtype=f32), utdt, zpad], axis=1)

    ab = (jnp.dot(hzb, wab_ref[...], preferred_element_type=f32)
          + bab_ref[...])
    v1rep = jnp.concatenate([v1] * _DYN, axis=1)
    p_ab = (ab * v1rep).astype(bf16)
    zfull = jnp.dot(p_ab, sab_ref[...],
                    preferred_element_type=f32)

    v2 = zfull + ushift
    cd = (jnp.dot(hzb, wcd_ref[...], preferred_element_type=f32)
          + bcd_ref[...])
    v2rep = jnp.concatenate([v2] * _NOBS, axis=1)
    p_cd = (cd * v2rep).astype(bf16)
    yt = jnp.dot(p_cd, scd_ref[...],
                 preferred_element_type=f32)

    z_ref[...] = zfull[:, :_DYN]
    y_ref[...] = yt


@jax.jit
def kernel(z_dyn, z_static, dt, ut, W1, b1, W2, b2, W3, b3,
           Wa, ba, Wb, bb, Wc, bc, Wd, bd):
    B = z_dyn.shape[0]
    f32 = jnp.float32
    bf16 = jnp.bfloat16

    W1d = W1[:_DYN].astype(bf16)
    W1s = W1[_DYN:_DYN + _STAT].astype(bf16)
    W1t = W1[_DYN + _STAT:]

    def interleave(Wx, bx, Wy, by, rows):
        padw = jnp.zeros((_HZ, rows, _PAD), dtype=f32)
        W = jnp.concatenate(
            [Wx.reshape(_HZ, rows, _DYN), Wy.reshape(_HZ, rows, _U), padw],
            axis=2).reshape(_HZ, rows * _LANE)
        padb = jnp.zeros((rows, _PAD), dtype=f32)
        bvec = jnp.concatenate(
            [bx.reshape(rows, _DYN), by.reshape(rows, _U), padb],
            axis=1).reshape(1, rows * _LANE)
        return W.astype(bf16), bvec

    Wab, bab = interleave(Wa, ba, Wb, bb, _DYN)
    Wcd, bcd = interleave(Wc, bc, Wd, bd, _NOBS)

    sab = (jnp.arange(_DYN * _LANE)[:, None] // _LANE
           == jnp.arange(_LANE)[None, :]).astype(bf16)
    scd = (jnp.arange(_NOBS * _LANE)[:, None] // _LANE
           == jnp.arange(_NOBS)[None, :]).astype(bf16)

    grid = (B // _TB,)
    row_spec = lambda n: pl.BlockSpec((_TB, n), lambda i: (i, 0))
    w_spec = lambda shp: pl.BlockSpec(shp, lambda i: (0,) * len(shp))

    z_next, yt = pl.pallas_call(
        _fused_body,
        grid=grid,
        in_specs=[
            row_spec(_DYN),
            row_spec(_STAT),
            row_spec(1),
            row_spec(_U),
            w_spec((_DYN, _H1)), w_spec((_STAT, _H1)), w_spec((1, _H1)),
            w_spec((1, _H1)),
            w_spec((_H1, _H2)), w_spec((1, _H2)),
            w_spec((_H2, _HZ)), w_spec((1, _HZ)),
            w_spec((_HZ, _DYN * _LANE)), w_spec((1, _DYN * _LANE)),
            w_spec((_HZ, _NOBS * _LANE)), w_spec((1, _NOBS * _LANE)),
            w_spec((_DYN * _LANE, _LANE)),
            w_spec((_NOBS * _LANE, _NOBS)),
        ],
        out_specs=[row_spec(_DYN), row_spec(_NOBS)],
        out_shape=[
            jax.ShapeDtypeStruct((B, _DYN), f32),
            jax.ShapeDtypeStruct((B, _NOBS), f32),
        ],
        compiler_params=pltpu.CompilerParams(
            dimension_semantics=("arbitrary",)),
    )(z_dyn, z_static, dt, ut,
      W1d, W1s, W1t, b1.reshape(1, -1),
      W2.astype(bf16), b2.reshape(1, -1),
      W3.astype(bf16), b3.reshape(1, -1),
      Wab, bab, Wcd, bcd, sab, scd)

    return (z_next, yt)


# speedup vs baseline: 1.5651x; 1.1024x over previous
import jax
import jax.numpy as jnp
from jax.experimental import pallas as pl
from jax.experimental.pallas import tpu as pltpu

_DYN = 96
_STAT = 32
_U = 8
_NOBS = 13
_HZ = 128
_H1 = 200
_H2 = 200
_LANE = 128

_TB = 256


def _seg_sum_matrix(n_cols, period, n_out):
    return (jnp.arange(n_cols)[:, None] // period
            == jnp.arange(n_out)[None, :]).astype(jnp.bfloat16)


def _fused_body(zd_ref, zs_ref, dt_ref, ut_ref,
                w1d_ref, w1s_ref, w1t_ref, b1_ref,
                w2_ref, b2_ref, w3_ref, b3_ref,
                wa_ref, ba_ref, wb_ref, bb_ref,
                wc_ref, bc_ref, wd_ref, bd_ref,
                sa_ref, sb_ref, sc_ref, sd_ref,
                z_ref, y_ref):
    bf16 = jnp.bfloat16
    f32 = jnp.float32
    zd = zd_ref[...]
    dt = dt_ref[...]
    h = (jnp.dot(zd.astype(bf16), w1d_ref[...], preferred_element_type=f32)
         + jnp.dot(zs_ref[...].astype(bf16), w1s_ref[...],
                   preferred_element_type=f32)
         + dt * w1t_ref[...] + b1_ref[...])
    h = jnp.maximum(h, 0.0)
    h = jnp.maximum(
        jnp.dot(h.astype(bf16), w2_ref[...], preferred_element_type=f32)
        + b2_ref[...], 0.0)
    hz = (jnp.dot(h.astype(bf16), w3_ref[...], preferred_element_type=f32)
          + b3_ref[...])
    hzb = hz.astype(bf16)

    utdt = ut_ref[...] * dt
    u128 = jnp.concatenate([utdt] * 16, axis=1)

    ab = (jnp.dot(hzb, wa_ref[...], preferred_element_type=f32)
          + ba_ref[...])
    z384 = jnp.concatenate([zd] * 4, axis=1)
    p_a = jnp.concatenate(
        [ab[:, k * 384:(k + 1) * 384] * z384 for k in range(24)],
        axis=1).astype(bf16)
    z_part = jnp.dot(p_a, sa_ref[...], preferred_element_type=f32)

    bt = (jnp.dot(hzb, wb_ref[...], preferred_element_type=f32)
          + bb_ref[...])
    p_b = jnp.concatenate(
        [bt[:, k * 128:(k + 1) * 128] * u128 for k in range(6)],
        axis=1).astype(bf16)
    zfull = z_part + jnp.dot(p_b, sb_ref[...],
                             preferred_element_type=f32)

    zn384 = jnp.concatenate([zfull[:, :_DYN]] * 4, axis=1)

    ct = (jnp.dot(hzb, wc_ref[...], preferred_element_type=f32)
          + bc_ref[...])
    p_c = jnp.concatenate(
        [ct[:, k * 384:(k + 1) * 384] * zn384 for k in range(3)]
        + [ct[:, 1152:1248] * zfull[:, :_DYN]],
        axis=1).astype(bf16)
    yt = jnp.dot(p_c, sc_ref[...], preferred_element_type=f32)

    dtv = (jnp.dot(hzb, wd_ref[...], preferred_element_type=f32)
           + bd_ref[...])
    p_d = (dtv * u128).astype(bf16)
    yt = yt + jnp.dot(p_d, sd_ref[...], preferred_element_type=f32)

    z_ref[...] = zfull[:, :_DYN]
    y_ref[...] = yt


@jax.jit
def kernel(z_dyn, z_static, dt, ut, W1, b1, W2, b2, W3, b3,
           Wa, ba, Wb, bb, Wc, bc, Wd, bd):
    B = z_dyn.shape[0]
    f32 = jnp.float32
    bf16 = jnp.bfloat16

    W1d = W1[:_DYN].astype(bf16)
    W1s = W1[_DYN:_DYN + _STAT].astype(bf16)
    W1t = W1[_DYN + _STAT:]
    Wd_p = jnp.concatenate(
        [Wd, jnp.zeros((_HZ, _LANE - _NOBS * _U), f32)], axis=1)
    bd_p = jnp.concatenate(
        [bd, jnp.zeros((_LANE - _NOBS * _U,), f32)])

    sa = _seg_sum_matrix(_DYN * _DYN, _DYN, _LANE)
    sb = _seg_sum_matrix(_DYN * _U, _U, _LANE)
    sc = _seg_sum_matrix(_NOBS * _DYN, _DYN, _NOBS)
    sd = _seg_sum_matrix(_LANE, _U, _NOBS)

    grid = (B // _TB,)
    row_spec = lambda n: pl.BlockSpec((_TB, n), lambda i: (i, 0))
    w_spec = lambda shp: pl.BlockSpec(shp, lambda i: (0,) * len(shp))

    z_next, yt = pl.pallas_call(
        _fused_body,
        grid=grid,
        in_specs=[
            row_spec(_DYN),
            row_spec(_STAT),
            row_spec(1),
            row_spec(_U),
            w_spec((_DYN, _H1)), w_spec((_STAT, _H1)), w_spec((1, _H1)),
            w_spec((1, _H1)),
            w_spec((_H1, _H2)), w_spec((1, _H2)),
            w_spec((_H2, _HZ)), w_spec((1, _HZ)),
            w_spec((_HZ, _DYN * _DYN)), w_spec((1, _DYN * _DYN)),
            w_spec((_HZ, _DYN * _U)), w_spec((1, _DYN * _U)),
            w_spec((_HZ, _NOBS * _DYN)), w_spec((1, _NOBS * _DYN)),
            w_spec((_HZ, _LANE)), w_spec((1, _LANE)),
            w_spec((_DYN * _DYN, _LANE)),
            w_spec((_DYN * _U, _LANE)),
            w_spec((_NOBS * _DYN, _NOBS)),
            w_spec((_LANE, _NOBS)),
        ],
        out_specs=[row_spec(_DYN), row_spec(_NOBS)],
        out_shape=[
            jax.ShapeDtypeStruct((B, _DYN), f32),
            jax.ShapeDtypeStruct((B, _NOBS), f32),
        ],
        compiler_params=pltpu.CompilerParams(
            dimension_semantics=("arbitrary",)),
    )(z_dyn, z_static, dt, ut,
      W1d, W1s, W1t, b1.reshape(1, -1),
      W2.astype(bf16), b2.reshape(1, -1),
      W3.astype(bf16), b3.reshape(1, -1),
      Wa.astype(bf16), ba.reshape(1, -1),
      Wb.astype(bf16), bb.reshape(1, -1),
      Wc.astype(bf16), bc.reshape(1, -1),
      Wd_p.astype(bf16), bd_p.reshape(1, -1),
      sa, sb, sc, sd)

    return (z_next, yt)


# speedup vs baseline: 1.6318x; 1.0426x over previous
import jax
import jax.numpy as jnp
from jax.experimental import pallas as pl
from jax.experimental.pallas import tpu as pltpu

_DYN = 96
_STAT = 32
_U = 8
_NOBS = 13
_HZ = 128
_H1 = 200
_H2 = 200
_LANE = 128

_TB = 256


def _seg_sum_matrix(n_cols, period, n_out):
    return (jnp.arange(n_cols)[:, None] // period
            == jnp.arange(n_out)[None, :]).astype(jnp.bfloat16)


def _fused_body(zd_ref, zs_ref, dt_ref, ut_ref,
                w1d_ref, w1s_ref, w1t_ref, b1_ref,
                w2_ref, b2_ref, w3_ref, b3_ref,
                wa_ref, ba_ref, wb_ref, bb_ref,
                wc_ref, bc_ref, wd_ref, bd_ref,
                sa_ref, sb_ref, sc_ref, sd_ref,
                z_ref, y_ref,
                w1d_bf, w1s_bf, w2_bf, w3_bf,
                wa_bf, wb_bf, wc_bf, wd_bf):
    bf16 = jnp.bfloat16
    f32 = jnp.float32

    @pl.when(pl.program_id(0) == 0)
    def _cast_weights():
        w1d_bf[...] = w1d_ref[...].astype(bf16)
        w1s_bf[...] = w1s_ref[...].astype(bf16)
        w2_bf[...] = w2_ref[...].astype(bf16)
        w3_bf[...] = w3_ref[...].astype(bf16)
        wa_bf[...] = wa_ref[...].astype(bf16)
        wb_bf[...] = wb_ref[...].astype(bf16)
        wc_bf[...] = wc_ref[...].astype(bf16)
        wd_bf[...] = wd_ref[...].astype(bf16)

    zd = zd_ref[...]
    dt = dt_ref[...]
    h = (jnp.dot(zd.astype(bf16), w1d_bf[...], preferred_element_type=f32)
         + jnp.dot(zs_ref[...].astype(bf16), w1s_bf[...],
                   preferred_element_type=f32)
         + dt * w1t_ref[...] + b1_ref[...])
    h = jnp.maximum(h, 0.0)
    h = jnp.maximum(
        jnp.dot(h.astype(bf16), w2_bf[...], preferred_element_type=f32)
        + b2_ref[...], 0.0)
    hz = (jnp.dot(h.astype(bf16), w3_bf[...], preferred_element_type=f32)
          + b3_ref[...])
    hzb = hz.astype(bf16)

    utdt = ut_ref[...] * dt
    u128 = jnp.concatenate([utdt] * 16, axis=1)

    ab = (jnp.dot(hzb, wa_bf[...], preferred_element_type=f32)
          + ba_ref[...])
    z384 = jnp.concatenate([zd] * 4, axis=1)
    p_a = jnp.concatenate(
        [ab[:, k * 384:(k + 1) * 384] * z384 for k in range(24)],
        axis=1).astype(bf16)
    z_part = jnp.dot(p_a, sa_ref[...], preferred_element_type=f32)

    bt = (jnp.dot(hzb, wb_bf[...], preferred_element_type=f32)
          + bb_ref[...])
    p_b = jnp.concatenate(
        [bt[:, k * 128:(k + 1) * 128] * u128 for k in range(6)],
        axis=1).astype(bf16)
    zfull = z_part + jnp.dot(p_b, sb_ref[...],
                             preferred_element_type=f32)

    zn384 = jnp.concatenate([zfull[:, :_DYN]] * 4, axis=1)

    ct = (jnp.dot(hzb, wc_bf[...], preferred_element_type=f32)
          + bc_ref[...])
    p_c = jnp.concatenate(
        [ct[:, k * 384:(k + 1) * 384] * zn384 for k in range(3)]
        + [ct[:, 1152:1248] * zfull[:, :_DYN]],
        axis=1).astype(bf16)
    yt = jnp.dot(p_c, sc_ref[...], preferred_element_type=f32)

    dtv = (jnp.dot(hzb, wd_bf[...], preferred_element_type=f32)
           + bd_ref[...])
    p_d = (dtv * u128[:, :_NOBS * _U]).astype(bf16)
    yt = yt + jnp.dot(p_d, sd_ref[...], preferred_element_type=f32)

    z_ref[...] = zfull[:, :_DYN]
    y_ref[...] = yt


@jax.jit
def kernel(z_dyn, z_static, dt, ut, W1, b1, W2, b2, W3, b3,
           Wa, ba, Wb, bb, Wc, bc, Wd, bd):
    B = z_dyn.shape[0]
    f32 = jnp.float32
    bf16 = jnp.bfloat16

    W1d = W1[:_DYN]
    W1s = W1[_DYN:_DYN + _STAT]
    W1t = W1[_DYN + _STAT:]

    sa = _seg_sum_matrix(_DYN * _DYN, _DYN, _LANE)
    sb = _seg_sum_matrix(_DYN * _U, _U, _LANE)
    sc = _seg_sum_matrix(_NOBS * _DYN, _DYN, _NOBS)
    sd = _seg_sum_matrix(_NOBS * _U, _U, _NOBS)

    grid = (B // _TB,)
    row_spec = lambda n: pl.BlockSpec((_TB, n), lambda i: (i, 0))
    w_spec = lambda shp: pl.BlockSpec(shp, lambda i: (0,) * len(shp))

    z_next, yt = pl.pallas_call(
        _fused_body,
        grid=grid,
        in_specs=[
            row_spec(_DYN),
            row_spec(_STAT),
            row_spec(1),
            row_spec(_U),
            w_spec((_DYN, _H1)), w_spec((_STAT, _H1)), w_spec((1, _H1)),
            w_spec((1, _H1)),
            w_spec((_H1, _H2)), w_spec((1, _H2)),
            w_spec((_H2, _HZ)), w_spec((1, _HZ)),
            w_spec((_HZ, _DYN * _DYN)), w_spec((1, _DYN * _DYN)),
            w_spec((_HZ, _DYN * _U)), w_spec((1, _DYN * _U)),
            w_spec((_HZ, _NOBS * _DYN)), w_spec((1, _NOBS * _DYN)),
            w_spec((_HZ, _NOBS * _U)), w_spec((1, _NOBS * _U)),
            w_spec((_DYN * _DYN, _LANE)),
            w_spec((_DYN * _U, _LANE)),
            w_spec((_NOBS * _DYN, _NOBS)),
            w_spec((_NOBS * _U, _NOBS)),
        ],
        out_specs=[row_spec(_DYN), row_spec(_NOBS)],
        out_shape=[
            jax.ShapeDtypeStruct((B, _DYN), f32),
            jax.ShapeDtypeStruct((B, _NOBS), f32),
        ],
        scratch_shapes=[
            pltpu.VMEM((_DYN, _H1), bf16),
            pltpu.VMEM((_STAT, _H1), bf16),
            pltpu.VMEM((_H1, _H2), bf16),
            pltpu.VMEM((_H2, _HZ), bf16),
            pltpu.VMEM((_HZ, _DYN * _DYN), bf16),
            pltpu.VMEM((_HZ, _DYN * _U), bf16),
            pltpu.VMEM((_HZ, _NOBS * _DYN), bf16),
            pltpu.VMEM((_HZ, _NOBS * _U), bf16),
        ],
        compiler_params=pltpu.CompilerParams(
            dimension_semantics=("arbitrary",)),
    )(z_dyn, z_static, dt, ut,
      W1d, W1s, W1t, b1.reshape(1, -1),
      W2, b2.reshape(1, -1),
      W3, b3.reshape(1, -1),
      Wa, ba.reshape(1, -1),
      Wb, bb.reshape(1, -1),
      Wc, bc.reshape(1, -1),
      Wd, bd.reshape(1, -1),
      sa, sb, sc, sd)

    return (z_next, yt)
